# baseline (speedup 1.0000x reference)
import functools

import jax
import jax.numpy as jnp
from jax import lax
from jax.experimental import pallas as pl
from jax.experimental.pallas import tpu as pltpu
from jax.experimental.pallas import tpu_sc as plsc

_NC = 2
_NS = 16
_NW = _NC * _NS
_LANES = 16
_CHUNK = 128


def _mf_body(b_per_w, d_latent, idx_u_hbm, idx_i_hbm, eu_hbm, ei_hbm, out_hbm,
             idx_u_v, idx_i_v, u_rows, i_rows, out_v, sem):
    wid = lax.axis_index("s") * _NC + lax.axis_index("c")
    base = wid * b_per_w

    cp_u = pltpu.async_copy(idx_u_hbm.at[pl.ds(base, b_per_w)], idx_u_v, sem)
    cp_i = pltpu.async_copy(idx_i_hbm.at[pl.ds(base, b_per_w)], idx_i_v, sem)
    cp_u.wait()
    cp_i.wait()

    copies = []
    for k in range(b_per_w // _CHUNK):
        sl = pl.ds(k * _CHUNK, _CHUNK)
        copies.append(pltpu.async_copy(eu_hbm.at[idx_u_v.at[sl]], u_rows.at[sl], sem))
        copies.append(pltpu.async_copy(ei_hbm.at[idx_i_v.at[sl]], i_rows.at[sl], sem))
    for cp in copies:
        cp.wait()

    def group(g, carry):
        row = g * _LANES + lax.iota(jnp.int32, _LANES)
        acc = jnp.zeros((_LANES,), jnp.float32)
        for j in range(d_latent):
            col = jnp.full((_LANES,), j, jnp.int32)
            cu = plsc.load_gather(u_rows, [row, col])
            ci = plsc.load_gather(i_rows, [row, col])
            acc = acc + cu * ci
        sig = 1.0 / (1.0 + jnp.exp(-acc))
        out_v[pl.ds(g * _LANES, _LANES)] = sig
        return carry

    lax.fori_loop(0, b_per_w // _LANES, group, None)
    pltpu.sync_copy(out_v, out_hbm.at[pl.ds(base, b_per_w)])


def kernel(idx_u, idx_i, embeds_u, embeds_i):
    batch = idx_u.shape[0]
    d_latent = embeds_u.shape[1]
    b_per_w = batch // _NW
    mesh = plsc.VectorSubcoreMesh(core_axis_name="c", subcore_axis_name="s")
    mf = pl.kernel(
        functools.partial(_mf_body, b_per_w, d_latent),
        out_type=jax.ShapeDtypeStruct((batch,), jnp.float32),
        mesh=mesh,
        compiler_params=pltpu.CompilerParams(
            needs_layout_passes=False, use_tc_tiling_on_sc=False),
        scratch_types=[
            pltpu.VMEM((b_per_w,), jnp.int32),
            pltpu.VMEM((b_per_w,), jnp.int32),
            pltpu.VMEM((b_per_w, d_latent), jnp.float32),
            pltpu.VMEM((b_per_w, d_latent), jnp.float32),
            pltpu.VMEM((b_per_w,), jnp.float32),
            pltpu.SemaphoreType.DMA,
        ],
    )
    return mf(idx_u.astype(jnp.int32), idx_i.astype(jnp.int32), embeds_u, embeds_i)
